# Initial kernel scaffold; baseline (speedup 1.0000x reference)
import jax
import jax.numpy as jnp
from jax.experimental import pallas as pl


def kernel(source_vertices, source_faces, target_vertices, target_faces, face_probs):
    raise NotImplementedError("write your pallas kernel here")


# speedup vs baseline: 12.2931x; 12.2931x over previous
import functools

import jax
import jax.numpy as jnp
from jax import lax
from jax.experimental import pallas as pl
from jax.experimental.pallas import tpu as pltpu
from jax.experimental.pallas import tpu_sc as plsc

K = 5
NPF = 4

NSF = 2048
NTF = 4096
NSV = 5000
NTV = 8192
NPTS = NSF * NPF
SV_PAD = 5120

NW = 32
SB_CH = NSF // NW
TB_CH = NTF // NW
PT_CH = NPTS // NW
L = 16

TILE = 128
GRID = NPTS // TILE
FWD_STEPS = NSF // TILE


def _sc_gather_body(svx, svy, svz, tvx, tvy, tvz,
                    sf1, sf2, sf3, tf1, tf2, tf3,
                    w1, w2, w3, fp,
                    sbx_o, sby_o, sbz_o, tbx_o, tby_o, tbz_o,
                    px_o, py_o, pz_o, pfp_o,
                    svx_v, svy_v, svz_v, tvx_v, tvy_v, tvz_v,
                    sf1_v, sf2_v, sf3_v, tf1_v, tf2_v, tf3_v,
                    w1_v, w2_v, w3_v, fp_v,
                    sb_bx, sb_by, sb_bz, tb_bx, tb_by, tb_bz,
                    p_bx, p_by, p_bz, p_bf):
    wid = lax.axis_index("s") * 2 + lax.axis_index("c")
    pltpu.sync_copy(svx, svx_v)
    pltpu.sync_copy(svy, svy_v)
    pltpu.sync_copy(svz, svz_v)
    pltpu.sync_copy(tvx, tvx_v)
    pltpu.sync_copy(tvy, tvy_v)
    pltpu.sync_copy(tvz, tvz_v)
    pltpu.sync_copy(sf1.at[pl.ds(wid * SB_CH, SB_CH)], sf1_v)
    pltpu.sync_copy(sf2.at[pl.ds(wid * SB_CH, SB_CH)], sf2_v)
    pltpu.sync_copy(sf3.at[pl.ds(wid * SB_CH, SB_CH)], sf3_v)
    pltpu.sync_copy(tf1.at[pl.ds(wid * TB_CH, TB_CH)], tf1_v)
    pltpu.sync_copy(tf2.at[pl.ds(wid * TB_CH, TB_CH)], tf2_v)
    pltpu.sync_copy(tf3.at[pl.ds(wid * TB_CH, TB_CH)], tf3_v)
    pltpu.sync_copy(w1.at[pl.ds(wid * PT_CH, PT_CH)], w1_v)
    pltpu.sync_copy(w2.at[pl.ds(wid * PT_CH, PT_CH)], w2_v)
    pltpu.sync_copy(w3.at[pl.ds(wid * PT_CH, PT_CH)], w3_v)
    pltpu.sync_copy(fp.at[pl.ds(wid * SB_CH, SB_CH)], fp_v)

    third = jnp.float32(1.0 / 3.0)

    def bary_group(f1_v, f2_v, f3_v, vx, vy, vz, bx, by, bz, base):
        i1 = f1_v[pl.ds(base, L)]
        i2 = f2_v[pl.ds(base, L)]
        i3 = f3_v[pl.ds(base, L)]
        gx = (plsc.load_gather(vx, [i1]) + plsc.load_gather(vx, [i2])
              + plsc.load_gather(vx, [i3])) * third
        gy = (plsc.load_gather(vy, [i1]) + plsc.load_gather(vy, [i2])
              + plsc.load_gather(vy, [i3])) * third
        gz = (plsc.load_gather(vz, [i1]) + plsc.load_gather(vz, [i2])
              + plsc.load_gather(vz, [i3])) * third
        bx[pl.ds(base, L)] = gx
        by[pl.ds(base, L)] = gy
        bz[pl.ds(base, L)] = gz

    for g in range(SB_CH // L):
        bary_group(sf1_v, sf2_v, sf3_v, svx_v, svy_v, svz_v,
                   sb_bx, sb_by, sb_bz, g * L)
    for g in range(TB_CH // L):
        bary_group(tf1_v, tf2_v, tf3_v, tvx_v, tvy_v, tvz_v,
                   tb_bx, tb_by, tb_bz, g * L)

    iota = lax.iota(jnp.int32, L)
    for g in range(PT_CH // L):
        base = g * L
        lidx = lax.shift_right_logical(iota + base, 2)
        i1 = plsc.load_gather(sf1_v, [lidx])
        i2 = plsc.load_gather(sf2_v, [lidx])
        i3 = plsc.load_gather(sf3_v, [lidx])
        u1 = w1_v[pl.ds(base, L)]
        u2 = w2_v[pl.ds(base, L)]
        u3 = w3_v[pl.ds(base, L)]
        p_bx[pl.ds(base, L)] = (u1 * plsc.load_gather(svx_v, [i1])
                                + u2 * plsc.load_gather(svx_v, [i2])
                                + u3 * plsc.load_gather(svx_v, [i3]))
        p_by[pl.ds(base, L)] = (u1 * plsc.load_gather(svy_v, [i1])
                                + u2 * plsc.load_gather(svy_v, [i2])
                                + u3 * plsc.load_gather(svy_v, [i3]))
        p_bz[pl.ds(base, L)] = (u1 * plsc.load_gather(svz_v, [i1])
                                + u2 * plsc.load_gather(svz_v, [i2])
                                + u3 * plsc.load_gather(svz_v, [i3]))
        p_bf[pl.ds(base, L)] = plsc.load_gather(fp_v, [lidx])

    pltpu.sync_copy(sb_bx, sbx_o.at[pl.ds(wid * SB_CH, SB_CH)])
    pltpu.sync_copy(sb_by, sby_o.at[pl.ds(wid * SB_CH, SB_CH)])
    pltpu.sync_copy(sb_bz, sbz_o.at[pl.ds(wid * SB_CH, SB_CH)])
    pltpu.sync_copy(tb_bx, tbx_o.at[pl.ds(wid * TB_CH, TB_CH)])
    pltpu.sync_copy(tb_by, tby_o.at[pl.ds(wid * TB_CH, TB_CH)])
    pltpu.sync_copy(tb_bz, tbz_o.at[pl.ds(wid * TB_CH, TB_CH)])
    pltpu.sync_copy(p_bx, px_o.at[pl.ds(wid * PT_CH, PT_CH)])
    pltpu.sync_copy(p_by, py_o.at[pl.ds(wid * PT_CH, PT_CH)])
    pltpu.sync_copy(p_bz, pz_o.at[pl.ds(wid * PT_CH, PT_CH)])
    pltpu.sync_copy(p_bf, pfp_o.at[pl.ds(wid * PT_CH, PT_CH)])


_f32 = jnp.float32
_i32 = jnp.int32


def _make_sc_gather():
    return functools.partial(
        pl.kernel,
        mesh=plsc.VectorSubcoreMesh(core_axis_name="c", subcore_axis_name="s"),
        compiler_params=pltpu.CompilerParams(use_tc_tiling_on_sc=False,
                                             needs_layout_passes=False),
        out_type=[
        jax.ShapeDtypeStruct((NSF,), _f32),
        jax.ShapeDtypeStruct((NSF,), _f32),
        jax.ShapeDtypeStruct((NSF,), _f32),
        jax.ShapeDtypeStruct((NTF,), _f32),
        jax.ShapeDtypeStruct((NTF,), _f32),
        jax.ShapeDtypeStruct((NTF,), _f32),
        jax.ShapeDtypeStruct((NPTS,), _f32),
        jax.ShapeDtypeStruct((NPTS,), _f32),
        jax.ShapeDtypeStruct((NPTS,), _f32),
        jax.ShapeDtypeStruct((NPTS,), _f32),
    ],
    scratch_types=[
        pltpu.VMEM((SV_PAD,), _f32), pltpu.VMEM((SV_PAD,), _f32),
        pltpu.VMEM((SV_PAD,), _f32),
        pltpu.VMEM((NTV,), _f32), pltpu.VMEM((NTV,), _f32),
        pltpu.VMEM((NTV,), _f32),
        pltpu.VMEM((SB_CH,), _i32), pltpu.VMEM((SB_CH,), _i32),
        pltpu.VMEM((SB_CH,), _i32),
        pltpu.VMEM((TB_CH,), _i32), pltpu.VMEM((TB_CH,), _i32),
        pltpu.VMEM((TB_CH,), _i32),
        pltpu.VMEM((PT_CH,), _f32), pltpu.VMEM((PT_CH,), _f32),
        pltpu.VMEM((PT_CH,), _f32),
        pltpu.VMEM((SB_CH,), _f32),
        pltpu.VMEM((SB_CH,), _f32), pltpu.VMEM((SB_CH,), _f32),
        pltpu.VMEM((SB_CH,), _f32),
        pltpu.VMEM((TB_CH,), _f32), pltpu.VMEM((TB_CH,), _f32),
        pltpu.VMEM((TB_CH,), _f32),
        pltpu.VMEM((PT_CH,), _f32), pltpu.VMEM((PT_CH,), _f32),
        pltpu.VMEM((PT_CH,), _f32), pltpu.VMEM((PT_CH,), _f32),
        ],
    )(_sc_gather_body)


def _tc_loss_body(pts_ref, pfp_ref, sbr_ref, fpc_ref, tbc_ref, sbc_ref,
                  fpr_ref, out_ref):
    i = pl.program_id(0)

    @pl.when(i == 0)
    def _init():
        out_ref[...] = jnp.zeros((1, 1), jnp.float32)

    tb = tbc_ref[...]
    tbx, tby, tbz = tb[0:1, :], tb[1:2, :], tb[2:3, :]
    p = pts_ref[...]
    pxc, pyc, pzc = p[:, 0:1], p[:, 1:2], p[:, 2:3]

    dt = (pxc - tbx) ** 2
    dt = dt + (pyc - tby) ** 2
    dt = dt + (pzc - tbz) ** 2
    min_tgt = jnp.min(dt, axis=1, keepdims=True)

    sb = sbc_ref[...]
    sbx, sby, sbz = sb[0:1, :], sb[1:2, :], sb[2:3, :]
    ds = (pxc - sbx) ** 2
    ds = ds + (pyc - sby) ** 2
    ds = ds + (pzc - sbz) ** 2

    cols = lax.broadcasted_iota(jnp.int32, (TILE, NSF), 1)
    rows = lax.broadcasted_iota(jnp.int32, (TILE, 1), 0) + i * TILE
    selfidx = lax.shift_right_logical(rows, 2)
    fpb = fpr_ref[...]

    big = jnp.float32(3.0e38)
    sum_all = jnp.zeros((TILE, 1), jnp.float32)
    sum_first = jnp.zeros((TILE, 1), jnp.float32)
    self_prod = jnp.zeros((TILE, 1), jnp.float32)
    self_found = jnp.zeros((TILE, 1), jnp.bool_)
    d = ds
    for t in range(K + 1):
        m = jnp.min(d, axis=1, keepdims=True)
        idx = jnp.min(jnp.where(d <= m, cols, NSF), axis=1, keepdims=True)
        onehot = cols == idx
        fpm = jnp.sum(jnp.where(onehot, fpb, 0.0), axis=1, keepdims=True)
        prod = fpm * m
        sum_all = sum_all + prod
        if t < K:
            sum_first = sum_first + prod
        is_self = idx == selfidx
        self_found = jnp.logical_or(self_found, is_self)
        self_prod = self_prod + jnp.where(is_self, prod, 0.0)
        if t < K:
            d = jnp.where(onehot, big, d)
    mean_term = jnp.where(self_found, sum_all - self_prod,
                          sum_first) * jnp.float32(1.0 / K)

    pf = pfp_ref[...]
    rev = pf * min_tgt + (1.0 - pf) * mean_term
    out_ref[...] += jnp.sum(rev, axis=0, keepdims=True)

    @pl.when(i < FWD_STEPS)
    def _fwd():
        s = sbr_ref[...]
        sx, sy, sz = s[:, 0:1], s[:, 1:2], s[:, 2:3]
        df = (sx - tbx) ** 2
        df = df + (sy - tby) ** 2
        df = df + (sz - tbz) ** 2
        mf = jnp.min(df, axis=1, keepdims=True)
        out_ref[...] += jnp.sum(fpc_ref[...] * mf, axis=0, keepdims=True)


def _tc_loss(pts2, pfp2, sbr, fpc, tbc, sbc, fpr):
    return pl.pallas_call(
        _tc_loss_body,
        grid=(GRID,),
        in_specs=[
            pl.BlockSpec((TILE, 3), lambda i: (i, 0)),
            pl.BlockSpec((TILE, 1), lambda i: (i, 0)),
            pl.BlockSpec((TILE, 3), lambda i: (jnp.minimum(i, FWD_STEPS - 1), 0)),
            pl.BlockSpec((TILE, 1), lambda i: (jnp.minimum(i, FWD_STEPS - 1), 0)),
            pl.BlockSpec((3, NTF), lambda i: (0, 0)),
            pl.BlockSpec((3, NSF), lambda i: (0, 0)),
            pl.BlockSpec((1, NSF), lambda i: (0, 0)),
        ],
        out_specs=pl.BlockSpec((1, 1), lambda i: (0, 0)),
        out_shape=jax.ShapeDtypeStruct((1, 1), jnp.float32),
        compiler_params=pltpu.CompilerParams(
            dimension_semantics=("arbitrary",)),
    )(pts2, pfp2, sbr, fpc, tbc, sbc, fpr)


def kernel(source_vertices, source_faces, target_vertices, target_faces,
           face_probs):
    sv = source_vertices[0]
    tv = target_vertices[0]
    svx = jnp.pad(sv[:, 0], (0, SV_PAD - NSV))
    svy = jnp.pad(sv[:, 1], (0, SV_PAD - NSV))
    svz = jnp.pad(sv[:, 2], (0, SV_PAD - NSV))
    tvx, tvy, tvz = tv[:, 0], tv[:, 1], tv[:, 2]
    sf1 = source_faces[:, 0].astype(jnp.int32)
    sf2 = source_faces[:, 1].astype(jnp.int32)
    sf3 = source_faces[:, 2].astype(jnp.int32)
    tf1 = target_faces[0].astype(jnp.int32)
    tf2 = target_faces[1].astype(jnp.int32)
    tf3 = target_faces[2].astype(jnp.int32)
    fp = face_probs.astype(jnp.float32)

    rk = jax.random.key(42)
    r1 = jnp.sqrt(jax.random.uniform(jax.random.fold_in(rk, 0), (NSF, NPF),
                                     dtype=jnp.float32))
    r2 = jax.random.uniform(jax.random.fold_in(rk, 1), (NSF, NPF),
                            dtype=jnp.float32)
    w1 = (1.0 - r1).reshape(-1)
    w2 = (r1 * (1.0 - r2)).reshape(-1)
    w3 = (r1 * r2).reshape(-1)

    (sbx, sby, sbz, tbx, tby, tbz, px, py, pz, pfp) = _make_sc_gather()(
        svx, svy, svz, tvx, tvy, tvz, sf1, sf2, sf3, tf1, tf2, tf3,
        w1, w2, w3, fp)

    pts2 = jnp.stack([px, py, pz], axis=1)
    sbr = jnp.stack([sbx, sby, sbz], axis=1)
    tbc = jnp.stack([tbx, tby, tbz], axis=0)
    sbc = jnp.stack([sbx, sby, sbz], axis=0)

    out = _tc_loss(pts2, pfp[:, None], sbr, fp[:, None], tbc, sbc,
                   fp[None, :])
    return out[0, 0]


# speedup vs baseline: 17.5254x; 1.4256x over previous
import functools

import jax
import jax.numpy as jnp
from jax import lax
from jax.experimental import pallas as pl
from jax.experimental.pallas import tpu as pltpu
from jax.experimental.pallas import tpu_sc as plsc

K = 5
NPF = 4

NSF = 2048
NTF = 4096
NSV = 5000
NTV = 8192
NPTS = NSF * NPF
SV_PAD = 5120

NW = 32
SB_CH = NSF // NW
TB_CH = NTF // NW
PT_CH = NPTS // NW
L = 16

TILE = 256
GRID = NPTS // TILE
FWD_STEPS = NSF // TILE


def _sc_gather_body(svx, svy, svz, tvx, tvy, tvz,
                    sf1, sf2, sf3, tf1, tf2, tf3,
                    w1, w2, w3, fp,
                    sbx_o, sby_o, sbz_o, tbx_o, tby_o, tbz_o,
                    px_o, py_o, pz_o, pfp_o,
                    svx_v, svy_v, svz_v, tvx_v, tvy_v, tvz_v,
                    sf1_v, sf2_v, sf3_v, tf1_v, tf2_v, tf3_v,
                    w1_v, w2_v, w3_v, fp_v,
                    sb_bx, sb_by, sb_bz, tb_bx, tb_by, tb_bz,
                    p_bx, p_by, p_bz, p_bf):
    wid = lax.axis_index("s") * 2 + lax.axis_index("c")
    pltpu.sync_copy(svx, svx_v)
    pltpu.sync_copy(svy, svy_v)
    pltpu.sync_copy(svz, svz_v)
    pltpu.sync_copy(tvx, tvx_v)
    pltpu.sync_copy(tvy, tvy_v)
    pltpu.sync_copy(tvz, tvz_v)
    pltpu.sync_copy(sf1.at[pl.ds(wid * SB_CH, SB_CH)], sf1_v)
    pltpu.sync_copy(sf2.at[pl.ds(wid * SB_CH, SB_CH)], sf2_v)
    pltpu.sync_copy(sf3.at[pl.ds(wid * SB_CH, SB_CH)], sf3_v)
    pltpu.sync_copy(tf1.at[pl.ds(wid * TB_CH, TB_CH)], tf1_v)
    pltpu.sync_copy(tf2.at[pl.ds(wid * TB_CH, TB_CH)], tf2_v)
    pltpu.sync_copy(tf3.at[pl.ds(wid * TB_CH, TB_CH)], tf3_v)
    pltpu.sync_copy(w1.at[pl.ds(wid * PT_CH, PT_CH)], w1_v)
    pltpu.sync_copy(w2.at[pl.ds(wid * PT_CH, PT_CH)], w2_v)
    pltpu.sync_copy(w3.at[pl.ds(wid * PT_CH, PT_CH)], w3_v)
    pltpu.sync_copy(fp.at[pl.ds(wid * SB_CH, SB_CH)], fp_v)

    third = jnp.float32(1.0 / 3.0)

    def bary_group(f1_v, f2_v, f3_v, vx, vy, vz, bx, by, bz, base):
        i1 = f1_v[pl.ds(base, L)]
        i2 = f2_v[pl.ds(base, L)]
        i3 = f3_v[pl.ds(base, L)]
        gx = (plsc.load_gather(vx, [i1]) + plsc.load_gather(vx, [i2])
              + plsc.load_gather(vx, [i3])) * third
        gy = (plsc.load_gather(vy, [i1]) + plsc.load_gather(vy, [i2])
              + plsc.load_gather(vy, [i3])) * third
        gz = (plsc.load_gather(vz, [i1]) + plsc.load_gather(vz, [i2])
              + plsc.load_gather(vz, [i3])) * third
        bx[pl.ds(base, L)] = gx
        by[pl.ds(base, L)] = gy
        bz[pl.ds(base, L)] = gz

    for g in range(SB_CH // L):
        bary_group(sf1_v, sf2_v, sf3_v, svx_v, svy_v, svz_v,
                   sb_bx, sb_by, sb_bz, g * L)
    for g in range(TB_CH // L):
        bary_group(tf1_v, tf2_v, tf3_v, tvx_v, tvy_v, tvz_v,
                   tb_bx, tb_by, tb_bz, g * L)

    iota = lax.iota(jnp.int32, L)
    for g in range(PT_CH // L):
        base = g * L
        lidx = lax.shift_right_logical(iota + base, 2)
        i1 = plsc.load_gather(sf1_v, [lidx])
        i2 = plsc.load_gather(sf2_v, [lidx])
        i3 = plsc.load_gather(sf3_v, [lidx])
        u1 = w1_v[pl.ds(base, L)]
        u2 = w2_v[pl.ds(base, L)]
        u3 = w3_v[pl.ds(base, L)]
        p_bx[pl.ds(base, L)] = (u1 * plsc.load_gather(svx_v, [i1])
                                + u2 * plsc.load_gather(svx_v, [i2])
                                + u3 * plsc.load_gather(svx_v, [i3]))
        p_by[pl.ds(base, L)] = (u1 * plsc.load_gather(svy_v, [i1])
                                + u2 * plsc.load_gather(svy_v, [i2])
                                + u3 * plsc.load_gather(svy_v, [i3]))
        p_bz[pl.ds(base, L)] = (u1 * plsc.load_gather(svz_v, [i1])
                                + u2 * plsc.load_gather(svz_v, [i2])
                                + u3 * plsc.load_gather(svz_v, [i3]))
        p_bf[pl.ds(base, L)] = plsc.load_gather(fp_v, [lidx])

    pltpu.sync_copy(sb_bx, sbx_o.at[pl.ds(wid * SB_CH, SB_CH)])
    pltpu.sync_copy(sb_by, sby_o.at[pl.ds(wid * SB_CH, SB_CH)])
    pltpu.sync_copy(sb_bz, sbz_o.at[pl.ds(wid * SB_CH, SB_CH)])
    pltpu.sync_copy(tb_bx, tbx_o.at[pl.ds(wid * TB_CH, TB_CH)])
    pltpu.sync_copy(tb_by, tby_o.at[pl.ds(wid * TB_CH, TB_CH)])
    pltpu.sync_copy(tb_bz, tbz_o.at[pl.ds(wid * TB_CH, TB_CH)])
    pltpu.sync_copy(p_bx, px_o.at[pl.ds(wid * PT_CH, PT_CH)])
    pltpu.sync_copy(p_by, py_o.at[pl.ds(wid * PT_CH, PT_CH)])
    pltpu.sync_copy(p_bz, pz_o.at[pl.ds(wid * PT_CH, PT_CH)])
    pltpu.sync_copy(p_bf, pfp_o.at[pl.ds(wid * PT_CH, PT_CH)])


_f32 = jnp.float32
_i32 = jnp.int32


def _make_sc_gather():
    return functools.partial(
        pl.kernel,
        mesh=plsc.VectorSubcoreMesh(core_axis_name="c", subcore_axis_name="s"),
        compiler_params=pltpu.CompilerParams(use_tc_tiling_on_sc=False,
                                             needs_layout_passes=False),
        out_type=[
        jax.ShapeDtypeStruct((NSF,), _f32),
        jax.ShapeDtypeStruct((NSF,), _f32),
        jax.ShapeDtypeStruct((NSF,), _f32),
        jax.ShapeDtypeStruct((NTF,), _f32),
        jax.ShapeDtypeStruct((NTF,), _f32),
        jax.ShapeDtypeStruct((NTF,), _f32),
        jax.ShapeDtypeStruct((NPTS,), _f32),
        jax.ShapeDtypeStruct((NPTS,), _f32),
        jax.ShapeDtypeStruct((NPTS,), _f32),
        jax.ShapeDtypeStruct((NPTS,), _f32),
    ],
    scratch_types=[
        pltpu.VMEM((SV_PAD,), _f32), pltpu.VMEM((SV_PAD,), _f32),
        pltpu.VMEM((SV_PAD,), _f32),
        pltpu.VMEM((NTV,), _f32), pltpu.VMEM((NTV,), _f32),
        pltpu.VMEM((NTV,), _f32),
        pltpu.VMEM((SB_CH,), _i32), pltpu.VMEM((SB_CH,), _i32),
        pltpu.VMEM((SB_CH,), _i32),
        pltpu.VMEM((TB_CH,), _i32), pltpu.VMEM((TB_CH,), _i32),
        pltpu.VMEM((TB_CH,), _i32),
        pltpu.VMEM((PT_CH,), _f32), pltpu.VMEM((PT_CH,), _f32),
        pltpu.VMEM((PT_CH,), _f32),
        pltpu.VMEM((SB_CH,), _f32),
        pltpu.VMEM((SB_CH,), _f32), pltpu.VMEM((SB_CH,), _f32),
        pltpu.VMEM((SB_CH,), _f32),
        pltpu.VMEM((TB_CH,), _f32), pltpu.VMEM((TB_CH,), _f32),
        pltpu.VMEM((TB_CH,), _f32),
        pltpu.VMEM((PT_CH,), _f32), pltpu.VMEM((PT_CH,), _f32),
        pltpu.VMEM((PT_CH,), _f32), pltpu.VMEM((PT_CH,), _f32),
        ],
    )(_sc_gather_body)


def _tc_loss_body(pts_ref, pfp_ref, sbr_ref, fpc_ref, tbc_ref, sbc_ref,
                  fpr_ref, out_ref):
    i = pl.program_id(0)

    @pl.when(i == 0)
    def _init():
        out_ref[...] = jnp.zeros((1, 1), jnp.float32)

    f32 = jnp.float32

    tb = tbc_ref[...]
    tbx, tby, tbz = tb[0:1, :], tb[1:2, :], tb[2:3, :]
    tn = tbx * tbx + tby * tby + tbz * tbz
    sb = sbc_ref[...]
    sbx, sby, sbz = sb[0:1, :], sb[1:2, :], sb[2:3, :]
    sn = sbx * sbx + sby * sby + sbz * sbz

    p = pts_ref[...]
    pxc, pyc, pzc = p[:, 0:1], p[:, 1:2], p[:, 2:3]
    pn = pxc * pxc + pyc * pyc + pzc * pzc
    nx, ny, nz = -2.0 * pxc, -2.0 * pyc, -2.0 * pzc

    nsplit = 2
    half = NTF // nsplit
    mt = None
    for h in range(nsplit):
        s0, s1 = h * half, (h + 1) * half
        dth = (nx * tbx[:, s0:s1]
               + (ny * tby[:, s0:s1]
                  + (nz * tbz[:, s0:s1] + (pn + tn[:, s0:s1]))))
        mh = jnp.min(dth, axis=1, keepdims=True)
        mt = mh if mt is None else jnp.minimum(mt, mh)
    min_tgt = jnp.maximum(mt, 0.0)

    ds = nx * sbx + (ny * sby + (nz * sbz + (pn + sn)))

    big = jnp.float32(3.0e38)
    m = jnp.min(ds, axis=1, keepdims=True)
    thresholds = [m]
    for t in range(K):
        m = jnp.min(jnp.where(ds > m, ds, big), axis=1, keepdims=True)
        thresholds.append(m)
    m5, m6 = thresholds[K - 1], thresholds[K]

    cols = lax.broadcasted_iota(jnp.int32, (TILE, NSF), 1)
    rows = lax.broadcasted_iota(jnp.int32, (TILE, 1), 0) + i * TILE
    selfidx = lax.shift_right_logical(rows, 2)
    fpb = fpr_ref[...]

    pd = fpb * jnp.maximum(ds, 0.0)
    sel6 = ds <= m6
    selfhot = cols == selfidx
    prodsum6 = jnp.sum(jnp.where(sel6, pd, 0.0), axis=1, keepdims=True)
    prodsum5 = jnp.sum(jnp.where(ds <= m5, pd, 0.0), axis=1, keepdims=True)
    self_prod = jnp.sum(jnp.where(selfhot, pd, 0.0), axis=1, keepdims=True)
    d_self = jnp.sum(jnp.where(selfhot, ds, 0.0), axis=1, keepdims=True)
    self_found = d_self <= m6
    mean_term = jnp.where(self_found, prodsum6 - self_prod,
                          prodsum5) * jnp.float32(1.0 / K)

    pf = pfp_ref[...]
    rev = pf * min_tgt + (1.0 - pf) * mean_term
    out_ref[...] += jnp.sum(rev, axis=0, keepdims=True)

    @pl.when(i < FWD_STEPS)
    def _fwd():
        s = sbr_ref[...]
        sx, sy, sz = s[:, 0:1], s[:, 1:2], s[:, 2:3]
        fn = sx * sx + sy * sy + sz * sz
        df = (-2.0 * sx) * tbx + ((-2.0 * sy) * tby
                                  + ((-2.0 * sz) * tbz + (fn + tn)))
        mf = jnp.maximum(jnp.min(df, axis=1, keepdims=True), 0.0)
        out_ref[...] += jnp.sum(fpc_ref[...] * mf, axis=0, keepdims=True)


def _tc_loss(pts2, pfp2, sbr, fpc, tbc, sbc, fpr):
    return pl.pallas_call(
        _tc_loss_body,
        grid=(GRID,),
        in_specs=[
            pl.BlockSpec((TILE, 3), lambda i: (i, 0)),
            pl.BlockSpec((TILE, 1), lambda i: (i, 0)),
            pl.BlockSpec((TILE, 3), lambda i: (jnp.minimum(i, FWD_STEPS - 1), 0)),
            pl.BlockSpec((TILE, 1), lambda i: (jnp.minimum(i, FWD_STEPS - 1), 0)),
            pl.BlockSpec((3, NTF), lambda i: (0, 0)),
            pl.BlockSpec((3, NSF), lambda i: (0, 0)),
            pl.BlockSpec((1, NSF), lambda i: (0, 0)),
        ],
        out_specs=pl.BlockSpec((1, 1), lambda i: (0, 0)),
        out_shape=jax.ShapeDtypeStruct((1, 1), jnp.float32),
        compiler_params=pltpu.CompilerParams(
            dimension_semantics=("arbitrary",)),
    )(pts2, pfp2, sbr, fpc, tbc, sbc, fpr)


def kernel(source_vertices, source_faces, target_vertices, target_faces,
           face_probs):
    sv = source_vertices[0]
    tv = target_vertices[0]
    svx = jnp.pad(sv[:, 0], (0, SV_PAD - NSV))
    svy = jnp.pad(sv[:, 1], (0, SV_PAD - NSV))
    svz = jnp.pad(sv[:, 2], (0, SV_PAD - NSV))
    tvx, tvy, tvz = tv[:, 0], tv[:, 1], tv[:, 2]
    sf1 = source_faces[:, 0].astype(jnp.int32)
    sf2 = source_faces[:, 1].astype(jnp.int32)
    sf3 = source_faces[:, 2].astype(jnp.int32)
    tf1 = target_faces[0].astype(jnp.int32)
    tf2 = target_faces[1].astype(jnp.int32)
    tf3 = target_faces[2].astype(jnp.int32)
    fp = face_probs.astype(jnp.float32)

    rk = jax.random.key(42)
    r1 = jnp.sqrt(jax.random.uniform(jax.random.fold_in(rk, 0), (NSF, NPF),
                                     dtype=jnp.float32))
    r2 = jax.random.uniform(jax.random.fold_in(rk, 1), (NSF, NPF),
                            dtype=jnp.float32)
    w1 = (1.0 - r1).reshape(-1)
    w2 = (r1 * (1.0 - r2)).reshape(-1)
    w3 = (r1 * r2).reshape(-1)

    (sbx, sby, sbz, tbx, tby, tbz, px, py, pz, pfp) = _make_sc_gather()(
        svx, svy, svz, tvx, tvy, tvz, sf1, sf2, sf3, tf1, tf2, tf3,
        w1, w2, w3, fp)

    pts2 = jnp.stack([px, py, pz], axis=1)
    sbr = jnp.stack([sbx, sby, sbz], axis=1)
    tbc = jnp.stack([tbx, tby, tbz], axis=0)
    sbc = jnp.stack([sbx, sby, sbz], axis=0)

    out = _tc_loss(pts2, pfp[:, None], sbr, fp[:, None], tbc, sbc,
                   fp[None, :])
    return out[0, 0]


# speedup vs baseline: 18.5200x; 1.0568x over previous
import functools

import jax
import jax.numpy as jnp
import numpy as np
from jax import lax
from jax.experimental import pallas as pl
from jax.experimental.pallas import tpu as pltpu
from jax.experimental.pallas import tpu_sc as plsc

K = 5
NPF = 4

NSF = 2048
NTF = 4096
NSV = 5000
NTV = 8192
NPTS = NSF * NPF
SV_PAD = 5120

NW = 32
SB_CH = NSF // NW
TB_CH = NTF // NW
PT_CH = NPTS // NW
L = 16

TILE = 256
GRID = NPTS // TILE
FWD_STEPS = NSF // TILE


def _weights_impl():
    rk = jax.random.key(42)
    r1 = jnp.sqrt(jax.random.uniform(jax.random.fold_in(rk, 0),
                                     (NSF, NPF), dtype=jnp.float32))
    r2 = jax.random.uniform(jax.random.fold_in(rk, 1), (NSF, NPF),
                            dtype=jnp.float32)
    return ((1.0 - r1).reshape(-1), (r1 * (1.0 - r2)).reshape(-1),
            (r1 * r2).reshape(-1))


_wcache = []


def _get_weights():
    if not _wcache:
        try:
            with jax.default_device(jax.local_devices(backend="cpu")[0]):
                vals = jax.jit(_weights_impl)()
            _wcache.append(tuple(np.asarray(v) for v in vals))
        except Exception:
            _wcache.append(None)
    cached = _wcache[0]
    if cached is None:
        return _weights_impl()
    return tuple(jnp.asarray(v) for v in cached)


def _sc_gather_body(svx, svy, svz, tvx, tvy, tvz,
                    sf1, sf2, sf3, tf1, tf2, tf3,
                    w1, w2, w3, fp,
                    sbx_o, sby_o, sbz_o, tbx_o, tby_o, tbz_o,
                    px_o, py_o, pz_o, pfp_o,
                    svx_v, svy_v, svz_v, tvx_v, tvy_v, tvz_v,
                    sf1_v, sf2_v, sf3_v, tf1_v, tf2_v, tf3_v,
                    w1_v, w2_v, w3_v, fp_v,
                    sb_bx, sb_by, sb_bz, tb_bx, tb_by, tb_bz,
                    p_bx, p_by, p_bz, p_bf):
    wid = lax.axis_index("s") * 2 + lax.axis_index("c")
    pltpu.sync_copy(svx, svx_v)
    pltpu.sync_copy(svy, svy_v)
    pltpu.sync_copy(svz, svz_v)
    pltpu.sync_copy(tvx, tvx_v)
    pltpu.sync_copy(tvy, tvy_v)
    pltpu.sync_copy(tvz, tvz_v)
    pltpu.sync_copy(sf1.at[pl.ds(wid * SB_CH, SB_CH)], sf1_v)
    pltpu.sync_copy(sf2.at[pl.ds(wid * SB_CH, SB_CH)], sf2_v)
    pltpu.sync_copy(sf3.at[pl.ds(wid * SB_CH, SB_CH)], sf3_v)
    pltpu.sync_copy(tf1.at[pl.ds(wid * TB_CH, TB_CH)], tf1_v)
    pltpu.sync_copy(tf2.at[pl.ds(wid * TB_CH, TB_CH)], tf2_v)
    pltpu.sync_copy(tf3.at[pl.ds(wid * TB_CH, TB_CH)], tf3_v)
    pltpu.sync_copy(w1.at[pl.ds(wid * PT_CH, PT_CH)], w1_v)
    pltpu.sync_copy(w2.at[pl.ds(wid * PT_CH, PT_CH)], w2_v)
    pltpu.sync_copy(w3.at[pl.ds(wid * PT_CH, PT_CH)], w3_v)
    pltpu.sync_copy(fp.at[pl.ds(wid * SB_CH, SB_CH)], fp_v)

    third = jnp.float32(1.0 / 3.0)

    def bary_group(f1_v, f2_v, f3_v, vx, vy, vz, bx, by, bz, base):
        i1 = f1_v[pl.ds(base, L)]
        i2 = f2_v[pl.ds(base, L)]
        i3 = f3_v[pl.ds(base, L)]
        gx = (plsc.load_gather(vx, [i1]) + plsc.load_gather(vx, [i2])
              + plsc.load_gather(vx, [i3])) * third
        gy = (plsc.load_gather(vy, [i1]) + plsc.load_gather(vy, [i2])
              + plsc.load_gather(vy, [i3])) * third
        gz = (plsc.load_gather(vz, [i1]) + plsc.load_gather(vz, [i2])
              + plsc.load_gather(vz, [i3])) * third
        bx[pl.ds(base, L)] = gx
        by[pl.ds(base, L)] = gy
        bz[pl.ds(base, L)] = gz

    for g in range(SB_CH // L):
        bary_group(sf1_v, sf2_v, sf3_v, svx_v, svy_v, svz_v,
                   sb_bx, sb_by, sb_bz, g * L)
    for g in range(TB_CH // L):
        bary_group(tf1_v, tf2_v, tf3_v, tvx_v, tvy_v, tvz_v,
                   tb_bx, tb_by, tb_bz, g * L)

    iota = lax.iota(jnp.int32, L)
    for g in range(PT_CH // L):
        base = g * L
        lidx = lax.shift_right_logical(iota + base, 2)
        i1 = plsc.load_gather(sf1_v, [lidx])
        i2 = plsc.load_gather(sf2_v, [lidx])
        i3 = plsc.load_gather(sf3_v, [lidx])
        u1 = w1_v[pl.ds(base, L)]
        u2 = w2_v[pl.ds(base, L)]
        u3 = w3_v[pl.ds(base, L)]
        p_bx[pl.ds(base, L)] = (u1 * plsc.load_gather(svx_v, [i1])
                                + u2 * plsc.load_gather(svx_v, [i2])
                                + u3 * plsc.load_gather(svx_v, [i3]))
        p_by[pl.ds(base, L)] = (u1 * plsc.load_gather(svy_v, [i1])
                                + u2 * plsc.load_gather(svy_v, [i2])
                                + u3 * plsc.load_gather(svy_v, [i3]))
        p_bz[pl.ds(base, L)] = (u1 * plsc.load_gather(svz_v, [i1])
                                + u2 * plsc.load_gather(svz_v, [i2])
                                + u3 * plsc.load_gather(svz_v, [i3]))
        p_bf[pl.ds(base, L)] = plsc.load_gather(fp_v, [lidx])

    pltpu.sync_copy(sb_bx, sbx_o.at[pl.ds(wid * SB_CH, SB_CH)])
    pltpu.sync_copy(sb_by, sby_o.at[pl.ds(wid * SB_CH, SB_CH)])
    pltpu.sync_copy(sb_bz, sbz_o.at[pl.ds(wid * SB_CH, SB_CH)])
    pltpu.sync_copy(tb_bx, tbx_o.at[pl.ds(wid * TB_CH, TB_CH)])
    pltpu.sync_copy(tb_by, tby_o.at[pl.ds(wid * TB_CH, TB_CH)])
    pltpu.sync_copy(tb_bz, tbz_o.at[pl.ds(wid * TB_CH, TB_CH)])
    pltpu.sync_copy(p_bx, px_o.at[pl.ds(wid * PT_CH, PT_CH)])
    pltpu.sync_copy(p_by, py_o.at[pl.ds(wid * PT_CH, PT_CH)])
    pltpu.sync_copy(p_bz, pz_o.at[pl.ds(wid * PT_CH, PT_CH)])
    pltpu.sync_copy(p_bf, pfp_o.at[pl.ds(wid * PT_CH, PT_CH)])


_f32 = jnp.float32
_i32 = jnp.int32


def _make_sc_gather():
    return functools.partial(
        pl.kernel,
        mesh=plsc.VectorSubcoreMesh(core_axis_name="c", subcore_axis_name="s"),
        compiler_params=pltpu.CompilerParams(use_tc_tiling_on_sc=False,
                                             needs_layout_passes=False),
        out_type=[
        jax.ShapeDtypeStruct((NSF,), _f32),
        jax.ShapeDtypeStruct((NSF,), _f32),
        jax.ShapeDtypeStruct((NSF,), _f32),
        jax.ShapeDtypeStruct((NTF,), _f32),
        jax.ShapeDtypeStruct((NTF,), _f32),
        jax.ShapeDtypeStruct((NTF,), _f32),
        jax.ShapeDtypeStruct((NPTS,), _f32),
        jax.ShapeDtypeStruct((NPTS,), _f32),
        jax.ShapeDtypeStruct((NPTS,), _f32),
        jax.ShapeDtypeStruct((NPTS,), _f32),
    ],
    scratch_types=[
        pltpu.VMEM((SV_PAD,), _f32), pltpu.VMEM((SV_PAD,), _f32),
        pltpu.VMEM((SV_PAD,), _f32),
        pltpu.VMEM((NTV,), _f32), pltpu.VMEM((NTV,), _f32),
        pltpu.VMEM((NTV,), _f32),
        pltpu.VMEM((SB_CH,), _i32), pltpu.VMEM((SB_CH,), _i32),
        pltpu.VMEM((SB_CH,), _i32),
        pltpu.VMEM((TB_CH,), _i32), pltpu.VMEM((TB_CH,), _i32),
        pltpu.VMEM((TB_CH,), _i32),
        pltpu.VMEM((PT_CH,), _f32), pltpu.VMEM((PT_CH,), _f32),
        pltpu.VMEM((PT_CH,), _f32),
        pltpu.VMEM((SB_CH,), _f32),
        pltpu.VMEM((SB_CH,), _f32), pltpu.VMEM((SB_CH,), _f32),
        pltpu.VMEM((SB_CH,), _f32),
        pltpu.VMEM((TB_CH,), _f32), pltpu.VMEM((TB_CH,), _f32),
        pltpu.VMEM((TB_CH,), _f32),
        pltpu.VMEM((PT_CH,), _f32), pltpu.VMEM((PT_CH,), _f32),
        pltpu.VMEM((PT_CH,), _f32), pltpu.VMEM((PT_CH,), _f32),
        ],
    )(_sc_gather_body)


def _tc_loss_body(px_ref, py_ref, pz_ref, pfp_ref,
                  sxr_ref, syr_ref, szr_ref, fpc_ref,
                  tbx_ref, tby_ref, tbz_ref,
                  sbx_ref, sby_ref, sbz_ref, fpr_ref, out_ref):
    i = pl.program_id(0)

    @pl.when(i == 0)
    def _init():
        out_ref[...] = jnp.zeros((1, 1), jnp.float32)

    tbx = tbx_ref[...].reshape(1, NTF)
    tby = tby_ref[...].reshape(1, NTF)
    tbz = tbz_ref[...].reshape(1, NTF)
    tn = tbx * tbx + tby * tby + tbz * tbz
    sbx = sbx_ref[...].reshape(1, NSF)
    sby = sby_ref[...].reshape(1, NSF)
    sbz = sbz_ref[...].reshape(1, NSF)
    sn = sbx * sbx + sby * sby + sbz * sbz

    pxc = px_ref[...].reshape(TILE, 1)
    pyc = py_ref[...].reshape(TILE, 1)
    pzc = pz_ref[...].reshape(TILE, 1)
    pn = pxc * pxc + pyc * pyc + pzc * pzc
    nx, ny, nz = -2.0 * pxc, -2.0 * pyc, -2.0 * pzc

    nsplit = 2
    half = NTF // nsplit
    mt = None
    for h in range(nsplit):
        s0, s1 = h * half, (h + 1) * half
        dth = (nx * tbx[:, s0:s1]
               + (ny * tby[:, s0:s1]
                  + (nz * tbz[:, s0:s1] + (pn + tn[:, s0:s1]))))
        mh = jnp.min(dth, axis=1, keepdims=True)
        mt = mh if mt is None else jnp.minimum(mt, mh)
    min_tgt = jnp.maximum(mt, 0.0)

    ds = nx * sbx + (ny * sby + (nz * sbz + (pn + sn)))

    big = jnp.float32(3.0e38)
    m = jnp.min(ds, axis=1, keepdims=True)
    thresholds = [m]
    for t in range(K):
        m = jnp.min(jnp.where(ds > m, ds, big), axis=1, keepdims=True)
        thresholds.append(m)
    m5, m6 = thresholds[K - 1], thresholds[K]

    cols = lax.broadcasted_iota(jnp.int32, (TILE, NSF), 1)
    rows = lax.broadcasted_iota(jnp.int32, (TILE, 1), 0) + i * TILE
    selfidx = lax.shift_right_logical(rows, 2)
    fpb = fpr_ref[...].reshape(1, NSF)

    pd = fpb * jnp.maximum(ds, 0.0)
    sel6 = ds <= m6
    selfhot = cols == selfidx
    prodsum6 = jnp.sum(jnp.where(sel6, pd, 0.0), axis=1, keepdims=True)
    prodsum5 = jnp.sum(jnp.where(ds <= m5, pd, 0.0), axis=1, keepdims=True)
    self_prod = jnp.sum(jnp.where(selfhot, pd, 0.0), axis=1, keepdims=True)
    d_self = jnp.sum(jnp.where(selfhot, ds, 0.0), axis=1, keepdims=True)
    self_found = d_self <= m6
    mean_term = jnp.where(self_found, prodsum6 - self_prod,
                          prodsum5) * jnp.float32(1.0 / K)

    pf = pfp_ref[...].reshape(TILE, 1)
    rev = pf * min_tgt + (1.0 - pf) * mean_term
    out_ref[...] += jnp.sum(rev, axis=0, keepdims=True)

    @pl.when(i < FWD_STEPS)
    def _fwd():
        sx = sxr_ref[...].reshape(TILE, 1)
        sy = syr_ref[...].reshape(TILE, 1)
        sz = szr_ref[...].reshape(TILE, 1)
        fn = sx * sx + sy * sy + sz * sz
        df = (-2.0 * sx) * tbx + ((-2.0 * sy) * tby
                                  + ((-2.0 * sz) * tbz + (fn + tn)))
        mf = jnp.maximum(jnp.min(df, axis=1, keepdims=True), 0.0)
        fpcv = fpc_ref[...].reshape(TILE, 1)
        out_ref[...] += jnp.sum(fpcv * mf, axis=0, keepdims=True)


def _tc_loss(px, py, pz, pfp, sbx, sby, sbz, tbx, tby, tbz, fp):
    row = pl.BlockSpec((TILE,), lambda i: (i,))
    fwd_row = pl.BlockSpec((TILE,), lambda i: (jnp.minimum(i, FWD_STEPS - 1),))
    full_t = pl.BlockSpec((NTF,), lambda i: (0,))
    full_s = pl.BlockSpec((NSF,), lambda i: (0,))
    return pl.pallas_call(
        _tc_loss_body,
        grid=(GRID,),
        in_specs=[row, row, row, row,
                  fwd_row, fwd_row, fwd_row, fwd_row,
                  full_t, full_t, full_t,
                  full_s, full_s, full_s, full_s],
        out_specs=pl.BlockSpec((1, 1), lambda i: (0, 0)),
        out_shape=jax.ShapeDtypeStruct((1, 1), jnp.float32),
        compiler_params=pltpu.CompilerParams(
            dimension_semantics=("arbitrary",)),
    )(px, py, pz, pfp, sbx, sby, sbz, fp, tbx, tby, tbz,
      sbx, sby, sbz, fp)


def kernel(source_vertices, source_faces, target_vertices, target_faces,
           face_probs):
    sv = source_vertices[0]
    tv = target_vertices[0]
    svx = jnp.pad(sv[:, 0], (0, SV_PAD - NSV))
    svy = jnp.pad(sv[:, 1], (0, SV_PAD - NSV))
    svz = jnp.pad(sv[:, 2], (0, SV_PAD - NSV))
    tvx, tvy, tvz = tv[:, 0], tv[:, 1], tv[:, 2]
    sf1 = source_faces[:, 0].astype(jnp.int32)
    sf2 = source_faces[:, 1].astype(jnp.int32)
    sf3 = source_faces[:, 2].astype(jnp.int32)
    tf1 = target_faces[0].astype(jnp.int32)
    tf2 = target_faces[1].astype(jnp.int32)
    tf3 = target_faces[2].astype(jnp.int32)
    fp = face_probs.astype(jnp.float32)
    w1, w2, w3 = _get_weights()

    (sbx, sby, sbz, tbx, tby, tbz, px, py, pz, pfp) = _make_sc_gather()(
        svx, svy, svz, tvx, tvy, tvz, sf1, sf2, sf3, tf1, tf2, tf3,
        w1, w2, w3, fp)

    out = _tc_loss(px, py, pz, pfp, sbx, sby, sbz, tbx, tby, tbz, fp)
    return out[0, 0]


# speedup vs baseline: 20.0897x; 1.0848x over previous
import functools

import jax
import jax.numpy as jnp
import numpy as np
from jax import lax
from jax.experimental import pallas as pl
from jax.experimental.pallas import tpu as pltpu
from jax.experimental.pallas import tpu_sc as plsc

K = 5
NPF = 4

NSF = 2048
NTF = 4096
NSV = 5000
NTV = 8192
NPTS = NSF * NPF
SV_PAD = 5120

NW = 32
SB_CH = NSF // NW
TB_CH = NTF // NW
PT_CH = NPTS // NW
L = 16

TILE = 256
GRID = NPTS // TILE
FWD_STEPS = NSF // TILE


def _weights_impl():
    rk = jax.random.key(42)
    r1 = jnp.sqrt(jax.random.uniform(jax.random.fold_in(rk, 0),
                                     (NSF, NPF), dtype=jnp.float32))
    r2 = jax.random.uniform(jax.random.fold_in(rk, 1), (NSF, NPF),
                            dtype=jnp.float32)
    return ((1.0 - r1).reshape(-1), (r1 * (1.0 - r2)).reshape(-1),
            (r1 * r2).reshape(-1))


_wcache = []


def _get_weights():
    if not _wcache:
        try:
            with jax.default_device(jax.local_devices(backend="cpu")[0]):
                vals = jax.jit(_weights_impl)()
            _wcache.append(tuple(np.asarray(v) for v in vals))
        except Exception:
            _wcache.append(None)
    cached = _wcache[0]
    if cached is None:
        return _weights_impl()
    return tuple(jnp.asarray(v) for v in cached)


def _sc_gather_body(svx, svy, svz, tvx, tvy, tvz,
                    sf1, sf2, sf3, tf1, tf2, tf3,
                    w1, w2, w3, fp,
                    sbx_o, sby_o, sbz_o, tbx_o, tby_o, tbz_o,
                    px_o, py_o, pz_o, pfp_o,
                    svx_v, svy_v, svz_v, tvx_v, tvy_v, tvz_v,
                    sf1_v, sf2_v, sf3_v, tf1_v, tf2_v, tf3_v,
                    w1_v, w2_v, w3_v, fp_v,
                    sb_bx, sb_by, sb_bz, tb_bx, tb_by, tb_bz,
                    p_bx, p_by, p_bz, p_bf):
    wid = lax.axis_index("s") * 2 + lax.axis_index("c")
    pltpu.sync_copy(svx, svx_v)
    pltpu.sync_copy(svy, svy_v)
    pltpu.sync_copy(svz, svz_v)
    pltpu.sync_copy(tvx, tvx_v)
    pltpu.sync_copy(tvy, tvy_v)
    pltpu.sync_copy(tvz, tvz_v)
    pltpu.sync_copy(sf1.at[pl.ds(wid * SB_CH, SB_CH)], sf1_v)
    pltpu.sync_copy(sf2.at[pl.ds(wid * SB_CH, SB_CH)], sf2_v)
    pltpu.sync_copy(sf3.at[pl.ds(wid * SB_CH, SB_CH)], sf3_v)
    pltpu.sync_copy(tf1.at[pl.ds(wid * TB_CH, TB_CH)], tf1_v)
    pltpu.sync_copy(tf2.at[pl.ds(wid * TB_CH, TB_CH)], tf2_v)
    pltpu.sync_copy(tf3.at[pl.ds(wid * TB_CH, TB_CH)], tf3_v)
    pltpu.sync_copy(w1.at[pl.ds(wid * PT_CH, PT_CH)], w1_v)
    pltpu.sync_copy(w2.at[pl.ds(wid * PT_CH, PT_CH)], w2_v)
    pltpu.sync_copy(w3.at[pl.ds(wid * PT_CH, PT_CH)], w3_v)
    pltpu.sync_copy(fp.at[pl.ds(wid * SB_CH, SB_CH)], fp_v)

    third = jnp.float32(1.0 / 3.0)

    def bary_group(f1_v, f2_v, f3_v, vx, vy, vz, bx, by, bz, base):
        i1 = f1_v[pl.ds(base, L)]
        i2 = f2_v[pl.ds(base, L)]
        i3 = f3_v[pl.ds(base, L)]
        gx = (plsc.load_gather(vx, [i1]) + plsc.load_gather(vx, [i2])
              + plsc.load_gather(vx, [i3])) * third
        gy = (plsc.load_gather(vy, [i1]) + plsc.load_gather(vy, [i2])
              + plsc.load_gather(vy, [i3])) * third
        gz = (plsc.load_gather(vz, [i1]) + plsc.load_gather(vz, [i2])
              + plsc.load_gather(vz, [i3])) * third
        bx[pl.ds(base, L)] = gx
        by[pl.ds(base, L)] = gy
        bz[pl.ds(base, L)] = gz

    for g in range(SB_CH // L):
        bary_group(sf1_v, sf2_v, sf3_v, svx_v, svy_v, svz_v,
                   sb_bx, sb_by, sb_bz, g * L)
    for g in range(TB_CH // L):
        bary_group(tf1_v, tf2_v, tf3_v, tvx_v, tvy_v, tvz_v,
                   tb_bx, tb_by, tb_bz, g * L)

    iota = lax.iota(jnp.int32, L)
    for g in range(PT_CH // L):
        base = g * L
        lidx = lax.shift_right_logical(iota + base, 2)
        i1 = plsc.load_gather(sf1_v, [lidx])
        i2 = plsc.load_gather(sf2_v, [lidx])
        i3 = plsc.load_gather(sf3_v, [lidx])
        u1 = w1_v[pl.ds(base, L)]
        u2 = w2_v[pl.ds(base, L)]
        u3 = w3_v[pl.ds(base, L)]
        p_bx[pl.ds(base, L)] = (u1 * plsc.load_gather(svx_v, [i1])
                                + u2 * plsc.load_gather(svx_v, [i2])
                                + u3 * plsc.load_gather(svx_v, [i3]))
        p_by[pl.ds(base, L)] = (u1 * plsc.load_gather(svy_v, [i1])
                                + u2 * plsc.load_gather(svy_v, [i2])
                                + u3 * plsc.load_gather(svy_v, [i3]))
        p_bz[pl.ds(base, L)] = (u1 * plsc.load_gather(svz_v, [i1])
                                + u2 * plsc.load_gather(svz_v, [i2])
                                + u3 * plsc.load_gather(svz_v, [i3]))
        p_bf[pl.ds(base, L)] = plsc.load_gather(fp_v, [lidx])

    pltpu.sync_copy(sb_bx, sbx_o.at[pl.ds(wid * SB_CH, SB_CH)])
    pltpu.sync_copy(sb_by, sby_o.at[pl.ds(wid * SB_CH, SB_CH)])
    pltpu.sync_copy(sb_bz, sbz_o.at[pl.ds(wid * SB_CH, SB_CH)])
    pltpu.sync_copy(tb_bx, tbx_o.at[pl.ds(wid * TB_CH, TB_CH)])
    pltpu.sync_copy(tb_by, tby_o.at[pl.ds(wid * TB_CH, TB_CH)])
    pltpu.sync_copy(tb_bz, tbz_o.at[pl.ds(wid * TB_CH, TB_CH)])
    pltpu.sync_copy(p_bx, px_o.at[pl.ds(wid * PT_CH, PT_CH)])
    pltpu.sync_copy(p_by, py_o.at[pl.ds(wid * PT_CH, PT_CH)])
    pltpu.sync_copy(p_bz, pz_o.at[pl.ds(wid * PT_CH, PT_CH)])
    pltpu.sync_copy(p_bf, pfp_o.at[pl.ds(wid * PT_CH, PT_CH)])


_f32 = jnp.float32
_i32 = jnp.int32


def _make_sc_gather():
    return functools.partial(
        pl.kernel,
        mesh=plsc.VectorSubcoreMesh(core_axis_name="c", subcore_axis_name="s"),
        compiler_params=pltpu.CompilerParams(use_tc_tiling_on_sc=False,
                                             needs_layout_passes=False),
        out_type=[
        jax.ShapeDtypeStruct((NSF,), _f32),
        jax.ShapeDtypeStruct((NSF,), _f32),
        jax.ShapeDtypeStruct((NSF,), _f32),
        jax.ShapeDtypeStruct((NTF,), _f32),
        jax.ShapeDtypeStruct((NTF,), _f32),
        jax.ShapeDtypeStruct((NTF,), _f32),
        jax.ShapeDtypeStruct((NPTS,), _f32),
        jax.ShapeDtypeStruct((NPTS,), _f32),
        jax.ShapeDtypeStruct((NPTS,), _f32),
        jax.ShapeDtypeStruct((NPTS,), _f32),
    ],
    scratch_types=[
        pltpu.VMEM((SV_PAD,), _f32), pltpu.VMEM((SV_PAD,), _f32),
        pltpu.VMEM((SV_PAD,), _f32),
        pltpu.VMEM((NTV,), _f32), pltpu.VMEM((NTV,), _f32),
        pltpu.VMEM((NTV,), _f32),
        pltpu.VMEM((SB_CH,), _i32), pltpu.VMEM((SB_CH,), _i32),
        pltpu.VMEM((SB_CH,), _i32),
        pltpu.VMEM((TB_CH,), _i32), pltpu.VMEM((TB_CH,), _i32),
        pltpu.VMEM((TB_CH,), _i32),
        pltpu.VMEM((PT_CH,), _f32), pltpu.VMEM((PT_CH,), _f32),
        pltpu.VMEM((PT_CH,), _f32),
        pltpu.VMEM((SB_CH,), _f32),
        pltpu.VMEM((SB_CH,), _f32), pltpu.VMEM((SB_CH,), _f32),
        pltpu.VMEM((SB_CH,), _f32),
        pltpu.VMEM((TB_CH,), _f32), pltpu.VMEM((TB_CH,), _f32),
        pltpu.VMEM((TB_CH,), _f32),
        pltpu.VMEM((PT_CH,), _f32), pltpu.VMEM((PT_CH,), _f32),
        pltpu.VMEM((PT_CH,), _f32), pltpu.VMEM((PT_CH,), _f32),
        ],
    )(_sc_gather_body)


def _tc_loss_body(px_ref, py_ref, pz_ref, pfp_ref,
                  sxr_ref, syr_ref, szr_ref, fpc_ref,
                  tbx_ref, tby_ref, tbz_ref,
                  sbx_ref, sby_ref, sbz_ref, fpr_ref, out_ref):
    i = pl.program_id(0)

    @pl.when(i == 0)
    def _init():
        out_ref[...] = jnp.zeros((1, 1), jnp.float32)

    tbx = tbx_ref[...].reshape(1, NTF)
    tby = tby_ref[...].reshape(1, NTF)
    tbz = tbz_ref[...].reshape(1, NTF)
    tn = tbx * tbx + tby * tby + tbz * tbz
    sbx = sbx_ref[...].reshape(1, NSF)
    sby = sby_ref[...].reshape(1, NSF)
    sbz = sbz_ref[...].reshape(1, NSF)
    sn = sbx * sbx + sby * sby + sbz * sbz

    pxc = px_ref[...].reshape(TILE, 1)
    pyc = py_ref[...].reshape(TILE, 1)
    pzc = pz_ref[...].reshape(TILE, 1)
    pn = pxc * pxc + pyc * pyc + pzc * pzc
    nx, ny, nz = -2.0 * pxc, -2.0 * pyc, -2.0 * pzc

    bf16 = jnp.bfloat16
    px16 = pxc.astype(bf16)
    py16 = pyc.astype(bf16)
    pz16 = pzc.astype(bf16)
    nsplit = 2
    half = NTF // nsplit
    mt = None
    for h in range(nsplit):
        s0, s1 = h * half, (h + 1) * half
        dxh = px16 - tbx[:, s0:s1].astype(bf16)
        dyh = py16 - tby[:, s0:s1].astype(bf16)
        dzh = pz16 - tbz[:, s0:s1].astype(bf16)
        dth = dxh * dxh + (dyh * dyh + dzh * dzh)
        mh = jnp.min(dth, axis=1, keepdims=True)
        mt = mh if mt is None else jnp.minimum(mt, mh)
    min_tgt = jnp.maximum(mt.astype(jnp.float32), 0.0)

    ds = nx * sbx + (ny * sby + (nz * sbz + (pn + sn)))

    big = jnp.float32(3.0e38)
    m = jnp.min(ds, axis=1, keepdims=True)
    thresholds = [m]
    for t in range(K):
        m = jnp.min(jnp.where(ds > m, ds, big), axis=1, keepdims=True)
        thresholds.append(m)
    m5, m6 = thresholds[K - 1], thresholds[K]

    cols = lax.broadcasted_iota(jnp.int32, (TILE, NSF), 1)
    rows = lax.broadcasted_iota(jnp.int32, (TILE, 1), 0) + i * TILE
    selfidx = lax.shift_right_logical(rows, 2)
    fpb = fpr_ref[...].reshape(1, NSF)

    pd = fpb * jnp.maximum(ds, 0.0)
    sel6 = ds <= m6
    selfhot = cols == selfidx
    prodsum6 = jnp.sum(jnp.where(sel6, pd, 0.0), axis=1, keepdims=True)
    prodsum5 = jnp.sum(jnp.where(ds <= m5, pd, 0.0), axis=1, keepdims=True)
    self_prod = jnp.sum(jnp.where(selfhot, pd, 0.0), axis=1, keepdims=True)
    d_self = jnp.sum(jnp.where(selfhot, ds, 0.0), axis=1, keepdims=True)
    self_found = d_self <= m6
    mean_term = jnp.where(self_found, prodsum6 - self_prod,
                          prodsum5) * jnp.float32(1.0 / K)

    pf = pfp_ref[...].reshape(TILE, 1)
    rev = pf * min_tgt + (1.0 - pf) * mean_term
    out_ref[...] += jnp.sum(rev, axis=0, keepdims=True)

    @pl.when(i < FWD_STEPS)
    def _fwd():
        sx = sxr_ref[...].reshape(TILE, 1)
        sy = syr_ref[...].reshape(TILE, 1)
        sz = szr_ref[...].reshape(TILE, 1)
        fn = sx * sx + sy * sy + sz * sz
        df = (-2.0 * sx) * tbx + ((-2.0 * sy) * tby
                                  + ((-2.0 * sz) * tbz + (fn + tn)))
        mf = jnp.maximum(jnp.min(df, axis=1, keepdims=True), 0.0)
        fpcv = fpc_ref[...].reshape(TILE, 1)
        out_ref[...] += jnp.sum(fpcv * mf, axis=0, keepdims=True)


def _tc_loss(px, py, pz, pfp, sbx, sby, sbz, tbx, tby, tbz, fp):
    row = pl.BlockSpec((TILE,), lambda i: (i,))
    fwd_row = pl.BlockSpec((TILE,), lambda i: (jnp.minimum(i, FWD_STEPS - 1),))
    full_t = pl.BlockSpec((NTF,), lambda i: (0,))
    full_s = pl.BlockSpec((NSF,), lambda i: (0,))
    return pl.pallas_call(
        _tc_loss_body,
        grid=(GRID,),
        in_specs=[row, row, row, row,
                  fwd_row, fwd_row, fwd_row, fwd_row,
                  full_t, full_t, full_t,
                  full_s, full_s, full_s, full_s],
        out_specs=pl.BlockSpec((1, 1), lambda i: (0, 0)),
        out_shape=jax.ShapeDtypeStruct((1, 1), jnp.float32),
        compiler_params=pltpu.CompilerParams(
            dimension_semantics=("arbitrary",)),
    )(px, py, pz, pfp, sbx, sby, sbz, fp, tbx, tby, tbz,
      sbx, sby, sbz, fp)


def kernel(source_vertices, source_faces, target_vertices, target_faces,
           face_probs):
    sv = source_vertices[0]
    tv = target_vertices[0]
    svx = jnp.pad(sv[:, 0], (0, SV_PAD - NSV))
    svy = jnp.pad(sv[:, 1], (0, SV_PAD - NSV))
    svz = jnp.pad(sv[:, 2], (0, SV_PAD - NSV))
    tvx, tvy, tvz = tv[:, 0], tv[:, 1], tv[:, 2]
    sf1 = source_faces[:, 0].astype(jnp.int32)
    sf2 = source_faces[:, 1].astype(jnp.int32)
    sf3 = source_faces[:, 2].astype(jnp.int32)
    tf1 = target_faces[0].astype(jnp.int32)
    tf2 = target_faces[1].astype(jnp.int32)
    tf3 = target_faces[2].astype(jnp.int32)
    fp = face_probs.astype(jnp.float32)
    w1, w2, w3 = _get_weights()

    (sbx, sby, sbz, tbx, tby, tbz, px, py, pz, pfp) = _make_sc_gather()(
        svx, svy, svz, tvx, tvy, tvz, sf1, sf2, sf3, tf1, tf2, tf3,
        w1, w2, w3, fp)

    out = _tc_loss(px, py, pz, pfp, sbx, sby, sbz, tbx, tby, tbz, fp)
    return out[0, 0]


# speedup vs baseline: 20.5756x; 1.0242x over previous
import functools

import jax
import jax.numpy as jnp
import numpy as np
from jax import lax
from jax.experimental import pallas as pl
from jax.experimental.pallas import tpu as pltpu
from jax.experimental.pallas import tpu_sc as plsc

K = 5
NPF = 4

NSF = 2048
NTF = 4096
NSV = 5000
NTV = 8192
NPTS = NSF * NPF
SV_PAD = 5120

NW = 32
SB_CH = NSF // NW
TB_CH = NTF // NW
PT_CH = NPTS // NW
L = 16

TILE = 256
GRID = NPTS // TILE
FWD_STEPS = NSF // TILE


def _weights_impl():
    rk = jax.random.key(42)
    r1 = jnp.sqrt(jax.random.uniform(jax.random.fold_in(rk, 0),
                                     (NSF, NPF), dtype=jnp.float32))
    r2 = jax.random.uniform(jax.random.fold_in(rk, 1), (NSF, NPF),
                            dtype=jnp.float32)
    return ((1.0 - r1).reshape(-1), (r1 * (1.0 - r2)).reshape(-1),
            (r1 * r2).reshape(-1))


_wcache = []


def _get_weights():
    if not _wcache:
        try:
            with jax.default_device(jax.local_devices(backend="cpu")[0]):
                vals = jax.jit(_weights_impl)()
            _wcache.append(tuple(np.asarray(v) for v in vals))
        except Exception:
            _wcache.append(None)
    cached = _wcache[0]
    if cached is None:
        return _weights_impl()
    return tuple(jnp.asarray(v) for v in cached)


def _sc_gather_body(svx, svy, svz, tvx, tvy, tvz,
                    sf1, sf2, sf3, tf1, tf2, tf3,
                    w1, w2, w3, fp,
                    sbx_o, sby_o, sbz_o, tbx_o, tby_o, tbz_o,
                    px_o, py_o, pz_o, pfp_o,
                    svx_v, svy_v, svz_v, tvx_v, tvy_v, tvz_v,
                    sf1_v, sf2_v, sf3_v, tf1_v, tf2_v, tf3_v,
                    w1_v, w2_v, w3_v, fp_v,
                    sb_bx, sb_by, sb_bz, tb_bx, tb_by, tb_bz,
                    p_bx, p_by, p_bz, p_bf):
    wid = lax.axis_index("s") * 2 + lax.axis_index("c")
    pltpu.sync_copy(svx, svx_v)
    pltpu.sync_copy(svy, svy_v)
    pltpu.sync_copy(svz, svz_v)
    pltpu.sync_copy(tvx, tvx_v)
    pltpu.sync_copy(tvy, tvy_v)
    pltpu.sync_copy(tvz, tvz_v)
    pltpu.sync_copy(sf1.at[pl.ds(wid * SB_CH, SB_CH)], sf1_v)
    pltpu.sync_copy(sf2.at[pl.ds(wid * SB_CH, SB_CH)], sf2_v)
    pltpu.sync_copy(sf3.at[pl.ds(wid * SB_CH, SB_CH)], sf3_v)
    pltpu.sync_copy(tf1.at[pl.ds(wid * TB_CH, TB_CH)], tf1_v)
    pltpu.sync_copy(tf2.at[pl.ds(wid * TB_CH, TB_CH)], tf2_v)
    pltpu.sync_copy(tf3.at[pl.ds(wid * TB_CH, TB_CH)], tf3_v)
    pltpu.sync_copy(w1.at[pl.ds(wid * PT_CH, PT_CH)], w1_v)
    pltpu.sync_copy(w2.at[pl.ds(wid * PT_CH, PT_CH)], w2_v)
    pltpu.sync_copy(w3.at[pl.ds(wid * PT_CH, PT_CH)], w3_v)
    pltpu.sync_copy(fp.at[pl.ds(wid * SB_CH, SB_CH)], fp_v)

    third = jnp.float32(1.0 / 3.0)

    def bary_group(f1_v, f2_v, f3_v, vx, vy, vz, bx, by, bz, base):
        i1 = f1_v[pl.ds(base, L)]
        i2 = f2_v[pl.ds(base, L)]
        i3 = f3_v[pl.ds(base, L)]
        gx = (plsc.load_gather(vx, [i1]) + plsc.load_gather(vx, [i2])
              + plsc.load_gather(vx, [i3])) * third
        gy = (plsc.load_gather(vy, [i1]) + plsc.load_gather(vy, [i2])
              + plsc.load_gather(vy, [i3])) * third
        gz = (plsc.load_gather(vz, [i1]) + plsc.load_gather(vz, [i2])
              + plsc.load_gather(vz, [i3])) * third
        bx[pl.ds(base, L)] = gx
        by[pl.ds(base, L)] = gy
        bz[pl.ds(base, L)] = gz

    for g in range(SB_CH // L):
        bary_group(sf1_v, sf2_v, sf3_v, svx_v, svy_v, svz_v,
                   sb_bx, sb_by, sb_bz, g * L)
    for g in range(TB_CH // L):
        bary_group(tf1_v, tf2_v, tf3_v, tvx_v, tvy_v, tvz_v,
                   tb_bx, tb_by, tb_bz, g * L)

    iota = lax.iota(jnp.int32, L)
    for g in range(PT_CH // L):
        base = g * L
        lidx = lax.shift_right_logical(iota + base, 2)
        i1 = plsc.load_gather(sf1_v, [lidx])
        i2 = plsc.load_gather(sf2_v, [lidx])
        i3 = plsc.load_gather(sf3_v, [lidx])
        u1 = w1_v[pl.ds(base, L)]
        u2 = w2_v[pl.ds(base, L)]
        u3 = w3_v[pl.ds(base, L)]
        p_bx[pl.ds(base, L)] = (u1 * plsc.load_gather(svx_v, [i1])
                                + u2 * plsc.load_gather(svx_v, [i2])
                                + u3 * plsc.load_gather(svx_v, [i3]))
        p_by[pl.ds(base, L)] = (u1 * plsc.load_gather(svy_v, [i1])
                                + u2 * plsc.load_gather(svy_v, [i2])
                                + u3 * plsc.load_gather(svy_v, [i3]))
        p_bz[pl.ds(base, L)] = (u1 * plsc.load_gather(svz_v, [i1])
                                + u2 * plsc.load_gather(svz_v, [i2])
                                + u3 * plsc.load_gather(svz_v, [i3]))
        p_bf[pl.ds(base, L)] = plsc.load_gather(fp_v, [lidx])

    pltpu.sync_copy(sb_bx, sbx_o.at[pl.ds(wid * SB_CH, SB_CH)])
    pltpu.sync_copy(sb_by, sby_o.at[pl.ds(wid * SB_CH, SB_CH)])
    pltpu.sync_copy(sb_bz, sbz_o.at[pl.ds(wid * SB_CH, SB_CH)])
    pltpu.sync_copy(tb_bx, tbx_o.at[pl.ds(wid * TB_CH, TB_CH)])
    pltpu.sync_copy(tb_by, tby_o.at[pl.ds(wid * TB_CH, TB_CH)])
    pltpu.sync_copy(tb_bz, tbz_o.at[pl.ds(wid * TB_CH, TB_CH)])
    pltpu.sync_copy(p_bx, px_o.at[pl.ds(wid * PT_CH, PT_CH)])
    pltpu.sync_copy(p_by, py_o.at[pl.ds(wid * PT_CH, PT_CH)])
    pltpu.sync_copy(p_bz, pz_o.at[pl.ds(wid * PT_CH, PT_CH)])
    pltpu.sync_copy(p_bf, pfp_o.at[pl.ds(wid * PT_CH, PT_CH)])


_f32 = jnp.float32
_i32 = jnp.int32


def _make_sc_gather():
    return functools.partial(
        pl.kernel,
        mesh=plsc.VectorSubcoreMesh(core_axis_name="c", subcore_axis_name="s"),
        compiler_params=pltpu.CompilerParams(use_tc_tiling_on_sc=False,
                                             needs_layout_passes=False),
        out_type=[
        jax.ShapeDtypeStruct((NSF,), _f32),
        jax.ShapeDtypeStruct((NSF,), _f32),
        jax.ShapeDtypeStruct((NSF,), _f32),
        jax.ShapeDtypeStruct((NTF,), _f32),
        jax.ShapeDtypeStruct((NTF,), _f32),
        jax.ShapeDtypeStruct((NTF,), _f32),
        jax.ShapeDtypeStruct((NPTS,), _f32),
        jax.ShapeDtypeStruct((NPTS,), _f32),
        jax.ShapeDtypeStruct((NPTS,), _f32),
        jax.ShapeDtypeStruct((NPTS,), _f32),
    ],
    scratch_types=[
        pltpu.VMEM((SV_PAD,), _f32), pltpu.VMEM((SV_PAD,), _f32),
        pltpu.VMEM((SV_PAD,), _f32),
        pltpu.VMEM((NTV,), _f32), pltpu.VMEM((NTV,), _f32),
        pltpu.VMEM((NTV,), _f32),
        pltpu.VMEM((SB_CH,), _i32), pltpu.VMEM((SB_CH,), _i32),
        pltpu.VMEM((SB_CH,), _i32),
        pltpu.VMEM((TB_CH,), _i32), pltpu.VMEM((TB_CH,), _i32),
        pltpu.VMEM((TB_CH,), _i32),
        pltpu.VMEM((PT_CH,), _f32), pltpu.VMEM((PT_CH,), _f32),
        pltpu.VMEM((PT_CH,), _f32),
        pltpu.VMEM((SB_CH,), _f32),
        pltpu.VMEM((SB_CH,), _f32), pltpu.VMEM((SB_CH,), _f32),
        pltpu.VMEM((SB_CH,), _f32),
        pltpu.VMEM((TB_CH,), _f32), pltpu.VMEM((TB_CH,), _f32),
        pltpu.VMEM((TB_CH,), _f32),
        pltpu.VMEM((PT_CH,), _f32), pltpu.VMEM((PT_CH,), _f32),
        pltpu.VMEM((PT_CH,), _f32), pltpu.VMEM((PT_CH,), _f32),
        ],
    )(_sc_gather_body)


def _tc_loss_body(px_ref, py_ref, pz_ref, pfp_ref,
                  sxr_ref, syr_ref, szr_ref, fpc_ref,
                  tbx_ref, tby_ref, tbz_ref,
                  sbx_ref, sby_ref, sbz_ref, fpr_ref, out_ref):
    i = pl.program_id(0)

    @pl.when(i == 0)
    def _init():
        out_ref[...] = jnp.zeros((1, 1), jnp.float32)

    bf16 = jnp.bfloat16
    tbx16 = tbx_ref[...].reshape(1, NTF).astype(bf16)
    tby16 = tby_ref[...].reshape(1, NTF).astype(bf16)
    tbz16 = tbz_ref[...].reshape(1, NTF).astype(bf16)
    sbx = sbx_ref[...].reshape(1, NSF)
    sby = sby_ref[...].reshape(1, NSF)
    sbz = sbz_ref[...].reshape(1, NSF)
    sn = sbx * sbx + sby * sby + sbz * sbz

    pxc = px_ref[...].reshape(TILE, 1)
    pyc = py_ref[...].reshape(TILE, 1)
    pzc = pz_ref[...].reshape(TILE, 1)
    pn = pxc * pxc + pyc * pyc + pzc * pzc
    nx, ny, nz = -2.0 * pxc, -2.0 * pyc, -2.0 * pzc

    px16 = pxc.astype(bf16)
    py16 = pyc.astype(bf16)
    pz16 = pzc.astype(bf16)
    nsplit = 2
    half = NTF // nsplit
    mt = None
    for h in range(nsplit):
        s0, s1 = h * half, (h + 1) * half
        dxh = px16 - tbx16[:, s0:s1]
        dyh = py16 - tby16[:, s0:s1]
        dzh = pz16 - tbz16[:, s0:s1]
        dth = dxh * dxh + (dyh * dyh + dzh * dzh)
        mh = jnp.min(dth, axis=1, keepdims=True)
        mt = mh if mt is None else jnp.minimum(mt, mh)
    min_tgt = jnp.maximum(mt.astype(jnp.float32), 0.0)

    ds = nx * sbx + (ny * sby + (nz * sbz + (pn + sn)))

    big = jnp.float32(3.0e38)
    m = jnp.min(ds, axis=1, keepdims=True)
    thresholds = [m]
    for t in range(K):
        m = jnp.min(jnp.where(ds > m, ds, big), axis=1, keepdims=True)
        thresholds.append(m)
    m5, m6 = thresholds[K - 1], thresholds[K]

    cols = lax.broadcasted_iota(jnp.int32, (TILE, NSF), 1)
    rows = lax.broadcasted_iota(jnp.int32, (TILE, 1), 0) + i * TILE
    selfidx = lax.shift_right_logical(rows, 2)
    fpb = fpr_ref[...].reshape(1, NSF)

    pd = fpb * jnp.maximum(ds, 0.0)
    sel6 = ds <= m6
    selfhot = cols == selfidx
    prodsum6 = jnp.sum(jnp.where(sel6, pd, 0.0), axis=1, keepdims=True)
    prodsum5 = jnp.sum(jnp.where(ds <= m5, pd, 0.0), axis=1, keepdims=True)
    self_prod = jnp.sum(jnp.where(selfhot, pd, 0.0), axis=1, keepdims=True)
    d_self = jnp.sum(jnp.where(selfhot, ds, 0.0), axis=1, keepdims=True)
    self_found = d_self <= m6
    mean_term = jnp.where(self_found, prodsum6 - self_prod,
                          prodsum5) * jnp.float32(1.0 / K)

    pf = pfp_ref[...].reshape(TILE, 1)
    rev = pf * min_tgt + (1.0 - pf) * mean_term
    out_ref[...] += jnp.sum(rev, axis=0, keepdims=True)

    @pl.when(i < FWD_STEPS)
    def _fwd():
        sx16 = sxr_ref[...].reshape(TILE, 1).astype(bf16)
        sy16 = syr_ref[...].reshape(TILE, 1).astype(bf16)
        sz16 = szr_ref[...].reshape(TILE, 1).astype(bf16)
        mf = None
        for h in range(nsplit):
            s0, s1 = h * half, (h + 1) * half
            fxh = sx16 - tbx16[:, s0:s1]
            fyh = sy16 - tby16[:, s0:s1]
            fzh = sz16 - tbz16[:, s0:s1]
            dfh = fxh * fxh + (fyh * fyh + fzh * fzh)
            mh = jnp.min(dfh, axis=1, keepdims=True)
            mf = mh if mf is None else jnp.minimum(mf, mh)
        mff = jnp.maximum(mf.astype(jnp.float32), 0.0)
        fpcv = fpc_ref[...].reshape(TILE, 1)
        out_ref[...] += jnp.sum(fpcv * mff, axis=0, keepdims=True)


def _tc_loss(px, py, pz, pfp, sbx, sby, sbz, tbx, tby, tbz, fp):
    row = pl.BlockSpec((TILE,), lambda i: (i,))
    fwd_row = pl.BlockSpec((TILE,), lambda i: (jnp.minimum(i, FWD_STEPS - 1),))
    full_t = pl.BlockSpec((NTF,), lambda i: (0,))
    full_s = pl.BlockSpec((NSF,), lambda i: (0,))
    return pl.pallas_call(
        _tc_loss_body,
        grid=(GRID,),
        in_specs=[row, row, row, row,
                  fwd_row, fwd_row, fwd_row, fwd_row,
                  full_t, full_t, full_t,
                  full_s, full_s, full_s, full_s],
        out_specs=pl.BlockSpec((1, 1), lambda i: (0, 0)),
        out_shape=jax.ShapeDtypeStruct((1, 1), jnp.float32),
        compiler_params=pltpu.CompilerParams(
            dimension_semantics=("arbitrary",)),
    )(px, py, pz, pfp, sbx, sby, sbz, fp, tbx, tby, tbz,
      sbx, sby, sbz, fp)


def kernel(source_vertices, source_faces, target_vertices, target_faces,
           face_probs):
    sv = source_vertices[0]
    tv = target_vertices[0]
    svx = jnp.pad(sv[:, 0], (0, SV_PAD - NSV))
    svy = jnp.pad(sv[:, 1], (0, SV_PAD - NSV))
    svz = jnp.pad(sv[:, 2], (0, SV_PAD - NSV))
    tvx, tvy, tvz = tv[:, 0], tv[:, 1], tv[:, 2]
    sf1 = source_faces[:, 0].astype(jnp.int32)
    sf2 = source_faces[:, 1].astype(jnp.int32)
    sf3 = source_faces[:, 2].astype(jnp.int32)
    tf1 = target_faces[0].astype(jnp.int32)
    tf2 = target_faces[1].astype(jnp.int32)
    tf3 = target_faces[2].astype(jnp.int32)
    fp = face_probs.astype(jnp.float32)
    w1, w2, w3 = _get_weights()

    (sbx, sby, sbz, tbx, tby, tbz, px, py, pz, pfp) = _make_sc_gather()(
        svx, svy, svz, tvx, tvy, tvz, sf1, sf2, sf3, tf1, tf2, tf3,
        w1, w2, w3, fp)

    out = _tc_loss(px, py, pz, pfp, sbx, sby, sbz, tbx, tby, tbz, fp)
    return out[0, 0]
